# baseline (speedup 1.0000x reference)
import jax
import jax.numpy as jnp
from jax import lax
from jax.experimental import pallas as pl
from jax.experimental.pallas import tpu as pltpu
from jax.experimental.pallas import tpu_sc as plsc

NC = 2
NS = 16
NW = NC * NS
BATCH = 4096
SEQ = 200
D = 64
SCALE = 8.0
LPW = BATCH // NW


def _body(seqT_hbm, table_hbm, p_hbm, out_hbm,
          p_v, i0, i1, r0, r1, t0, t1, is0, is1, g0, g1, w0, w1):
    wid = lax.axis_index("s") * NC + lax.axis_index("c")
    col0 = wid * LPW

    pltpu.sync_copy(p_hbm, p_v)

    idxs, rows, trans = [i0, i1], [r0, r1], [t0, t1]
    isems, gsems, wsems = [is0, is1], [g0, g1], [w0, w1]

    def idx_start(t, buf, sem):
        pltpu.async_copy(seqT_hbm.at[t, pl.ds(col0, LPW)], buf, sem)

    def idx_wait(t, buf, sem):
        pltpu.make_async_copy(seqT_hbm.at[t, pl.ds(col0, LPW)], buf, sem).wait()

    def g_start(ibuf, rbuf, sem):
        pltpu.async_copy(table_hbm.at[ibuf], rbuf, sem)

    def g_wait(ibuf, rbuf, sem):
        pltpu.make_async_copy(table_hbm.at[ibuf], rbuf, sem).wait()

    def wb_start(t, tbuf, sem):
        pltpu.async_copy(tbuf, out_hbm.at[t, pl.ds(0, 8), wid], sem)

    def wb_wait(t, tbuf, sem):
        pltpu.make_async_copy(tbuf, out_hbm.at[t, pl.ds(0, 8), wid], sem).wait()

    pltpu.sync_copy(seqT_hbm.at[0, pl.ds(col0, LPW)], i0)
    g_start(i0, r0, g0)
    idx_start(1, i1, is1)

    iota16 = jnp.arange(16, dtype=jnp.int32)

    def compute(t, ibuf, rbuf, tbuf):
        t16 = jnp.full((16,), t, jnp.int32)
        scales = []
        toks = []
        for c in range(LPW // 16):
            idx16 = ibuf[pl.ds(c * 16, 16)]
            scales.append(jnp.where(idx16 == 0, 0.0, SCALE).astype(jnp.float32))
            toks.append(iota16 + (c * 16))

        @plsc.parallel_loop(0, 64, 1, unroll=4)
        def rstep(i):
            r = i & 15
            j0 = i & ~15
            jv = (iota16 ^ r) + j0
            pj = plsc.load_gather(p_v, [t16, jv])
            jbv = jv >> 3
            jsv = jv & 7
            for c in range(LPW // 16):
                val = plsc.load_gather(rbuf, [toks[c], jv])
                plsc.store_scatter(tbuf, [jbv, jsv, toks[c]],
                                   val * scales[c] + pj)

    def step(s, carry):
        for r in range(2):
            t = 2 * s + r
            ibuf, rbuf, tbuf = idxs[r], rows[r], trans[r]
            isem, gsem, wsem = isems[r], gsems[r], wsems[r]
            nibuf, nrbuf = idxs[1 - r], rows[1 - r]

            g_wait(ibuf, rbuf, gsem)

            @pl.when(t + 1 < SEQ)
            def _():
                idx_wait(t + 1, nibuf, isems[1 - r])
                g_start(nibuf, nrbuf, gsems[1 - r])

            @pl.when(t + 2 < SEQ)
            def _():
                idx_start(t + 2, ibuf, isem)

            @pl.when(t >= 2)
            def _():
                wb_wait(t - 2, tbuf, wsem)

            compute(t, ibuf, rbuf, tbuf)
            wb_start(t, tbuf, wsem)
        return carry

    lax.fori_loop(0, SEQ // 2, step, 0)

    wb_wait(SEQ - 2, t0, w0)
    wb_wait(SEQ - 1, t1, w1)


def kernel(seq, table, p):
    mesh = plsc.VectorSubcoreMesh(core_axis_name="c", subcore_axis_name="s")
    f = pl.kernel(
        _body,
        out_type=jax.ShapeDtypeStruct((SEQ, D // 8, BATCH // 128, 8, 128),
                                      jnp.float32),
        mesh=mesh,
        compiler_params=pltpu.CompilerParams(needs_layout_passes=False,
                                             use_tc_tiling_on_sc=False),
        scratch_types=[
            pltpu.VMEM((SEQ, D), jnp.float32),
            pltpu.VMEM((LPW,), jnp.int32),
            pltpu.VMEM((LPW,), jnp.int32),
            pltpu.VMEM((LPW, D), jnp.float32),
            pltpu.VMEM((LPW, D), jnp.float32),
            pltpu.VMEM((D // 8, 8, 128), jnp.float32),
            pltpu.VMEM((D // 8, 8, 128), jnp.float32),
            pltpu.SemaphoreType.DMA,
            pltpu.SemaphoreType.DMA,
            pltpu.SemaphoreType.DMA,
            pltpu.SemaphoreType.DMA,
            pltpu.SemaphoreType.DMA,
            pltpu.SemaphoreType.DMA,
        ],
    )
    out5 = f(seq.T, table, p)
    return jnp.transpose(out5, (2, 4, 0, 1, 3)).reshape(BATCH, SEQ, D)


# speedup vs baseline: 1.2251x; 1.2251x over previous
import jax
import jax.numpy as jnp
from jax import lax
from jax.experimental import pallas as pl
from jax.experimental.pallas import tpu as pltpu
from jax.experimental.pallas import tpu_sc as plsc

NC = 2
NS = 16
NW = NC * NS
BATCH = 4096
SEQ = 200
D = 64
SCALE = 8.0
LPW = BATCH // NW


def _body(seqT_hbm, table_hbm, p_hbm, out_hbm,
          p_v, i0, i1, r0, r1, t0, t1, is0, is1, g0, g1, w0, w1):
    wid = lax.axis_index("s") * NC + lax.axis_index("c")
    col0 = wid * LPW

    pltpu.sync_copy(p_hbm, p_v)

    idxs, rows, trans = [i0, i1], [r0, r1], [t0, t1]
    isems, gsems, wsems = [is0, is1], [g0, g1], [w0, w1]

    def idx_start(t, buf, sem):
        pltpu.async_copy(seqT_hbm.at[t, pl.ds(col0, LPW)], buf, sem)

    def idx_wait(t, buf, sem):
        pltpu.make_async_copy(seqT_hbm.at[t, pl.ds(col0, LPW)], buf, sem).wait()

    def g_start(ibuf, rbuf, sem):
        pltpu.async_copy(table_hbm.at[ibuf], rbuf, sem)

    def g_wait(ibuf, rbuf, sem):
        pltpu.make_async_copy(table_hbm.at[ibuf], rbuf, sem).wait()

    def wb_start(t, tbuf, sem):
        pltpu.async_copy(tbuf, out_hbm.at[t, pl.ds(0, 8), wid], sem)

    def wb_wait(t, tbuf, sem):
        pltpu.make_async_copy(tbuf, out_hbm.at[t, pl.ds(0, 8), wid], sem).wait()

    pltpu.sync_copy(seqT_hbm.at[0, pl.ds(col0, LPW)], i0)
    g_start(i0, r0, g0)
    idx_start(1, i1, is1)

    iota16 = jnp.arange(16, dtype=jnp.int32)

    def compute(t, ibuf, rbuf, tbuf):
        t16 = jnp.full((16,), t, jnp.int32)
        scales = []
        toks = []
        for c in range(LPW // 16):
            idx16 = ibuf[pl.ds(c * 16, 16)]
            scales.append(jnp.where(idx16 == 0, 0.0, SCALE).astype(jnp.float32))
            toks.append(iota16 + (c * 16))

        @plsc.parallel_loop(0, 16, 1, unroll=2)
        def rstep(r):
            base = iota16 ^ r
            for j0 in range(0, D, 16):
                jv = base + j0
                pj = plsc.load_gather(p_v, [t16, jv])
                jbv = jv >> 3
                jsv = jv & 7
                for c in range(LPW // 16):
                    val = plsc.load_gather(rbuf, [toks[c], jv])
                    plsc.store_scatter(tbuf, [jbv, jsv, toks[c]],
                                       val * scales[c] + pj)

    def step(s, carry):
        for r in range(2):
            t = 2 * s + r
            ibuf, rbuf, tbuf = idxs[r], rows[r], trans[r]
            isem, gsem, wsem = isems[r], gsems[r], wsems[r]
            nibuf, nrbuf = idxs[1 - r], rows[1 - r]

            g_wait(ibuf, rbuf, gsem)

            @pl.when(t + 1 < SEQ)
            def _():
                idx_wait(t + 1, nibuf, isems[1 - r])
                g_start(nibuf, nrbuf, gsems[1 - r])

            @pl.when(t + 2 < SEQ)
            def _():
                idx_start(t + 2, ibuf, isem)

            @pl.when(t >= 2)
            def _():
                wb_wait(t - 2, tbuf, wsem)

            compute(t, ibuf, rbuf, tbuf)
            wb_start(t, tbuf, wsem)
        return carry

    lax.fori_loop(0, SEQ // 2, step, 0)

    wb_wait(SEQ - 2, t0, w0)
    wb_wait(SEQ - 1, t1, w1)


def kernel(seq, table, p):
    mesh = plsc.VectorSubcoreMesh(core_axis_name="c", subcore_axis_name="s")
    f = pl.kernel(
        _body,
        out_type=jax.ShapeDtypeStruct((SEQ, D // 8, BATCH // 128, 8, 128),
                                      jnp.float32),
        mesh=mesh,
        compiler_params=pltpu.CompilerParams(needs_layout_passes=False,
                                             use_tc_tiling_on_sc=False),
        scratch_types=[
            pltpu.VMEM((SEQ, D), jnp.float32),
            pltpu.VMEM((LPW,), jnp.int32),
            pltpu.VMEM((LPW,), jnp.int32),
            pltpu.VMEM((LPW, D), jnp.float32),
            pltpu.VMEM((LPW, D), jnp.float32),
            pltpu.VMEM((D // 8, 8, 128), jnp.float32),
            pltpu.VMEM((D // 8, 8, 128), jnp.float32),
            pltpu.SemaphoreType.DMA,
            pltpu.SemaphoreType.DMA,
            pltpu.SemaphoreType.DMA,
            pltpu.SemaphoreType.DMA,
            pltpu.SemaphoreType.DMA,
            pltpu.SemaphoreType.DMA,
        ],
    )
    out5 = f(seq.T, table, p)
    return jnp.transpose(out5, (2, 4, 0, 1, 3)).reshape(BATCH, SEQ, D)
